# Initial kernel scaffold; baseline (speedup 1.0000x reference)
import jax
import jax.numpy as jnp
from jax.experimental import pallas as pl


def kernel(features, codebook):
    raise NotImplementedError("write your pallas kernel here")


# speedup vs baseline: 1.1461x; 1.1461x over previous
import functools

import jax
import jax.numpy as jnp
from jax.experimental import pallas as pl
from jax.experimental.pallas import tpu as pltpu

BETA = 0.2
B, C, H, W = 8, 64, 64, 64
K, D = 1024, 64
N = B * H * W
BT = 512
NB = (H * W) // BT


def _vq_block(feat_ref, cb_ref, y2_ref, out_ref, loss_ref, acc_ref):
    b = pl.program_id(0)
    t = pl.program_id(1)
    x = feat_ref[...]
    cb = cb_ref[...]
    y2 = y2_ref[...]
    scores = y2 - 2.0 * jax.lax.dot_general(
        cb.astype(jnp.bfloat16), x.astype(jnp.bfloat16),
        (((1,), (0,)), ((), ())),
        preferred_element_type=jnp.float32)
    smin = jnp.min(scores, axis=0)
    iota_k = jax.lax.broadcasted_iota(jnp.int32, (K, BT), 0)
    idx = jnp.min(jnp.where(scores == smin[None, :], iota_k, K), axis=0)
    onehot = (jax.lax.broadcasted_iota(jnp.int32, (BT, K), 1)
              == idx[:, None]).astype(jnp.float32)
    out_ref[...] = jax.lax.dot_general(
        onehot, cb, (((1,), (0,)), ((), ())),
        preferred_element_type=jnp.float32,
        precision=jax.lax.Precision.HIGHEST)

    part = jnp.sum(smin) + jnp.sum(x * x)

    @pl.when((b == 0) & (t == 0))
    def _init():
        acc_ref[0] = 0.0

    acc_ref[0] += part

    @pl.when((b == B - 1) & (t == NB - 1))
    def _fin():
        loss_ref[...] = jnp.full((1, 1), acc_ref[0] * ((1.0 + BETA) / (N * D)),
                                 dtype=jnp.float32)


@functools.partial(jax.jit, static_argnames=("interpret",))
def kernel(features, codebook, interpret: bool = False):
    feat2d = features.reshape(B * C, H * W)
    y2 = jnp.sum(codebook * codebook, axis=1, keepdims=True)
    out, loss = pl.pallas_call(
        _vq_block,
        grid=(B, NB),
        in_specs=[
            pl.BlockSpec((C, BT), lambda b, t: (b, t)),
            pl.BlockSpec((K, D), lambda b, t: (0, 0)),
            pl.BlockSpec((K, 1), lambda b, t: (0, 0)),
        ],
        out_specs=[
            pl.BlockSpec((BT, D), lambda b, t: (b * NB + t, 0)),
            pl.BlockSpec((1, 1), lambda b, t: (0, 0)),
        ],
        out_shape=[
            jax.ShapeDtypeStruct((N, D), jnp.float32),
            jax.ShapeDtypeStruct((1, 1), jnp.float32),
        ],
        scratch_shapes=[pltpu.SMEM((1,), jnp.float32)],
        interpret=interpret,
    )(feat2d, codebook, y2)
    return out.reshape(B, C, H, W), loss[0, 0]


# speedup vs baseline: 1.9403x; 1.6930x over previous
import functools

import jax
import jax.numpy as jnp
from jax.experimental import pallas as pl
from jax.experimental.pallas import tpu as pltpu

BETA = 0.2
B, C, H, W = 8, 64, 64, 64
K, D = 1024, 64
N = B * H * W
BT = 512
NB = (H * W) // BT


def _vq_block(feat_ref, cbh_ref, cbl_ref, y2_ref, out_ref, loss_ref, acc_ref):
    b = pl.program_id(0)
    t = pl.program_id(1)
    x = feat_ref[...]
    cb_hi = cbh_ref[...]
    cb_lo = cbl_ref[...]
    y2 = y2_ref[...]
    scores = y2 - 2.0 * jax.lax.dot_general(
        cb_hi, x.astype(jnp.bfloat16), (((1,), (0,)), ((), ())),
        preferred_element_type=jnp.float32)
    smin = jnp.min(scores, axis=0)
    onehot = (scores == smin[None, :]).astype(jnp.bfloat16)
    out_ref[...] = (
        jax.lax.dot_general(onehot, cb_hi, (((0,), (0,)), ((), ())),
                            preferred_element_type=jnp.float32)
        + jax.lax.dot_general(onehot, cb_lo, (((0,), (0,)), ((), ())),
                              preferred_element_type=jnp.float32))

    part = jnp.sum(smin) + jnp.sum(x * x)

    @pl.when((b == 0) & (t == 0))
    def _init():
        acc_ref[0] = 0.0

    acc_ref[0] += part

    @pl.when((b == B - 1) & (t == NB - 1))
    def _fin():
        loss_ref[...] = jnp.full((1, 1), acc_ref[0] * ((1.0 + BETA) / (N * D)),
                                 dtype=jnp.float32)


@functools.partial(jax.jit, static_argnames=("interpret",))
def kernel(features, codebook, interpret: bool = False):
    feat2d = features.reshape(B * C, H * W)
    y2 = jnp.sum(codebook * codebook, axis=1, keepdims=True)
    cb_hi = codebook.astype(jnp.bfloat16)
    cb_lo = (codebook - cb_hi.astype(jnp.float32)).astype(jnp.bfloat16)
    out, loss = pl.pallas_call(
        _vq_block,
        grid=(B, NB),
        in_specs=[
            pl.BlockSpec((C, BT), lambda b, t: (b, t)),
            pl.BlockSpec((K, D), lambda b, t: (0, 0)),
            pl.BlockSpec((K, D), lambda b, t: (0, 0)),
            pl.BlockSpec((K, 1), lambda b, t: (0, 0)),
        ],
        out_specs=[
            pl.BlockSpec((BT, D), lambda b, t: (b * NB + t, 0)),
            pl.BlockSpec((1, 1), lambda b, t: (0, 0)),
        ],
        out_shape=[
            jax.ShapeDtypeStruct((N, D), jnp.float32),
            jax.ShapeDtypeStruct((1, 1), jnp.float32),
        ],
        scratch_shapes=[pltpu.SMEM((1,), jnp.float32)],
        interpret=interpret,
    )(feat2d, cb_hi, cb_lo, y2)
    return out.reshape(B, C, H, W), loss[0, 0]
